# baseline (speedup 1.0000x reference)
import jax
import jax.numpy as jnp
from jax import lax
from jax.experimental import pallas as pl
from jax.experimental.pallas import tpu as pltpu
from jax.experimental.pallas import tpu_sc as plsc

H, W, D = 16384, 20, 64
S = 2 * W
OUT_W = W * D + S
NC, NS = 2, 16
NW = NC * NS
RPW = H // NW
HS = RPW // 2


def _body(x_hbm, idxt_hbm, da_hbm, out_hbm, idx_v, rows0, rows1, rows2,
          da_v, sg0, sg1, sg2, sw0, sw1, sw2):
  wid = lax.axis_index("s") * NC + lax.axis_index("c")
  wbase = wid * RPW
  bufs = (rows0, rows1, rows2)
  gsems = (sg0, sg1, sg2)
  wsems = (sw0, sw1, sw2)
  pltpu.sync_copy(idxt_hbm.at[:, pl.ds(wbase, RPW)], idx_v)

  def gather(j, b):
    pltpu.async_copy(x_hbm.at[idx_v.at[j]], bufs[b], gsems[b])

  def wdesc(j, b):
    col = pl.multiple_of(j * D, D)
    return pltpu.make_async_copy(
        bufs[b], out_hbm.at[pl.ds(wbase, RPW), pl.ds(col, D)], wsems[b])

  gather(0, 0)
  gather(1, 1)
  for h in range(2):
    pltpu.sync_copy(da_hbm.at[pl.ds(wbase + h * HS, HS), :], da_v)
    pltpu.sync_copy(
        da_v, out_hbm.at[pl.ds(wbase + h * HS, HS), pl.ds(W * D, S)])

  def step(j, t):
    pltpu.make_async_copy(x_hbm.at[idx_v.at[j]], bufs[t], gsems[t]).wait()
    wdesc(j, t).start()
    t2 = (t + 2) % 3

    @pl.when(j >= 1)
    def _():
      wdesc(j - 1, t2).wait()

    gather(j + 2, t2)

  def trip(g, carry):
    for t in range(3):
      step(3 * g + t, t)
    return carry

  lax.fori_loop(0, (W - 2) // 3, trip, 0)
  pltpu.make_async_copy(x_hbm.at[idx_v.at[W - 2]], bufs[0], sg0).wait()
  wdesc(W - 2, 0).start()
  pltpu.make_async_copy(x_hbm.at[idx_v.at[W - 1]], bufs[1], sg1).wait()
  wdesc(W - 1, 1).start()
  wdesc(W - 3, 2).wait()
  wdesc(W - 2, 0).wait()
  wdesc(W - 1, 1).wait()


@jax.jit
def _run(x, idxt, da):
  mesh = plsc.VectorSubcoreMesh(core_axis_name="c", subcore_axis_name="s")
  return pl.kernel(
      _body,
      out_type=jax.ShapeDtypeStruct((H, OUT_W), jnp.float32),
      mesh=mesh,
      scratch_types=[
          pltpu.VMEM((W, RPW), jnp.int32),
          pltpu.VMEM((RPW, D), jnp.float32),
          pltpu.VMEM((RPW, D), jnp.float32),
          pltpu.VMEM((RPW, D), jnp.float32),
          pltpu.VMEM((HS, S), jnp.float32),
          pltpu.SemaphoreType.DMA,
          pltpu.SemaphoreType.DMA,
          pltpu.SemaphoreType.DMA,
          pltpu.SemaphoreType.DMA,
          pltpu.SemaphoreType.DMA,
          pltpu.SemaphoreType.DMA,
      ],
      compiler_params=pltpu.CompilerParams(use_tc_tiling_on_sc=False),
  )(x, idxt, da)


def kernel(x, idx, dis, angle_t):
  idxt = idx.astype(jnp.int32).T
  da = jnp.concatenate([dis, angle_t], axis=1)
  return _run(x, idxt, da)
